# Initial kernel scaffold; baseline (speedup 1.0000x reference)
import jax
import jax.numpy as jnp
from jax.experimental import pallas as pl


def kernel(pos_u, pos_v, neg_v, u_weight, v_weight):
    raise NotImplementedError("write your pallas kernel here")


# speedup vs baseline: 3.9620x; 3.9620x over previous
import functools

import jax
import jax.numpy as jnp
from jax import lax
from jax.experimental import pallas as pl
from jax.experimental.pallas import tpu as pltpu
from jax.experimental.pallas import tpu_sc as plsc

EMB_DIM = 64
BATCH = 16384
NEG = 20

NC = 2
NS = 16
L = 16
NW = NC * NS
PER_W = BATCH // NW
CHUNK = 64
NCHUNK = PER_W // CHUNK
NEG_SLAB = 128
NSLAB = CHUNK * NEG // NEG_SLAB

_LN2 = 0.6931472
_SQRT2 = 1.41421356


def _vlog(y):
    i = plsc.bitcast(y, jnp.int32)
    e = lax.shift_right_arithmetic(i, 23) - 127
    m = plsc.bitcast(
        (i & jnp.int32(0x007FFFFF)) | jnp.int32(0x3F800000), jnp.float32)
    big = m > _SQRT2
    m = jnp.where(big, m * 0.5, m)
    e = e + jnp.where(big, jnp.int32(1), jnp.int32(0))
    s = (m - 1.0) / (m + 1.0)
    z = s * s
    log_m = s * (2.0 + z * (0.66666667 + z * (0.4 + z * 0.2857143)))
    return e.astype(jnp.float32) * _LN2 + log_m


def _softplus_clipped(x):
    x = jnp.clip(x, -10.0, 10.0)
    return _vlog(1.0 + jnp.exp(x))


def _sg_body(pos_u_hbm, pos_v_hbm, neg_hbm, u_w, v_w, out_hbm,
             idxu_v, idxv_v, idxn_v, rows_u, rows_v, rows_n, acc_v, sem):
    wid = lax.axis_index("s") * NC + lax.axis_index("c")
    base = wid * PER_W
    iota = lax.iota(jnp.int32, L)

    def chunk_body(c, acc):
        cb = base + c * CHUNK
        pltpu.sync_copy(pos_u_hbm.at[pl.ds(cb, CHUNK)], idxu_v)
        pltpu.sync_copy(pos_v_hbm.at[pl.ds(cb, CHUNK)], idxv_v)
        pltpu.sync_copy(
            neg_hbm.at[pl.ds(cb * NEG, CHUNK * NEG)], idxn_v)
        cps = [
            pltpu.async_copy(u_w.at[idxu_v], rows_u, sem),
            pltpu.async_copy(v_w.at[idxv_v], rows_v, sem),
        ]
        for j in range(NSLAB):
            cps.append(pltpu.async_copy(
                v_w.at[idxn_v.at[pl.ds(j * NEG_SLAB, NEG_SLAB)]],
                rows_n.at[pl.ds(j * NEG_SLAB, NEG_SLAB)], sem))
        for cp in cps:
            cp.wait()

        for g in range(CHUNK // L):
            urows = iota + g * L
            nrows = iota * NEG + g * (L * NEG)

            def d_body(d, carry):
                accp, accns = carry
                dfull = jnp.broadcast_to(d, (L,))
                uu = plsc.load_gather(rows_u, [urows, dfull])
                vv = plsc.load_gather(rows_v, [urows, dfull])
                accp = accp + uu * vv
                accns = tuple(
                    accns[n]
                    + uu * plsc.load_gather(rows_n, [nrows + n, dfull])
                    for n in range(NEG))
                return accp, accns

            zero = jnp.zeros((L,), jnp.float32)
            accp, accns = lax.fori_loop(
                0, EMB_DIM, d_body, (zero, (zero,) * NEG))

            total = _softplus_clipped(-accp)
            for n in range(NEG):
                total = total + _softplus_clipped(accns[n])
            acc = acc + total
        return acc

    acc = lax.fori_loop(0, NCHUNK, chunk_body, jnp.zeros((L,), jnp.float32))
    acc_v[...] = acc
    pltpu.sync_copy(acc_v, out_hbm.at[wid])


@jax.jit
def _sg_call(pos_u, pos_v, neg2d, u_w, v_w):
    mesh = plsc.VectorSubcoreMesh(core_axis_name="c", subcore_axis_name="s")
    return pl.kernel(
        _sg_body,
        out_type=jax.ShapeDtypeStruct((NW, L), jnp.float32),
        mesh=mesh,
        scratch_types=[
            pltpu.VMEM((CHUNK,), jnp.int32),
            pltpu.VMEM((CHUNK,), jnp.int32),
            pltpu.VMEM((CHUNK * NEG,), jnp.int32),
            pltpu.VMEM((CHUNK, EMB_DIM), jnp.float32),
            pltpu.VMEM((CHUNK, EMB_DIM), jnp.float32),
            pltpu.VMEM((CHUNK * NEG, EMB_DIM), jnp.float32),
            pltpu.VMEM((L,), jnp.float32),
            pltpu.SemaphoreType.DMA,
        ],
        compiler_params=pltpu.CompilerParams(
            needs_layout_passes=False, use_tc_tiling_on_sc=False),
    )(pos_u, pos_v, neg2d, u_w, v_w)


def kernel(pos_u, pos_v, neg_v, u_weight, v_weight):
    neg_flat = neg_v.reshape(BATCH * NEG)
    partials = _sg_call(pos_u.astype(jnp.int32), pos_v.astype(jnp.int32),
                        neg_flat.astype(jnp.int32), u_weight, v_weight)
    return jnp.sum(partials) / BATCH
